# baseline (speedup 1.0000x reference)
import functools

import jax
import jax.numpy as jnp
from jax import lax
from jax.experimental import pallas as pl
from jax.experimental.pallas import tpu as pltpu
from jax.experimental.pallas import tpu_sc as plsc

_NC = 2
_NS = 16
_NW = _NC * _NS

_W0 = 64 // _NW
_NCHUNK = 32
_C1 = 128 // _NCHUNK


def _sc_copy(x_hbm, o_hbm, buf0, buf1, si0, si1, so0, so1):
    wid = lax.axis_index("s") * _NC + lax.axis_index("c")
    base = wid * _W0
    bufs = (buf0, buf1)
    in_sems = (si0, si1)
    out_sems = (so0, so1)

    def in_copy(j):
        b = j % 2
        return pltpu.make_async_copy(
            x_hbm.at[pl.ds(base, _W0), pl.ds(j * _C1, _C1)], bufs[b], in_sems[b]
        )

    def out_copy(j):
        b = j % 2
        return pltpu.make_async_copy(
            bufs[b], o_hbm.at[pl.ds(base, _W0), pl.ds(j * _C1, _C1)], out_sems[b]
        )

    in_copy(0).start()
    for j in range(_NCHUNK):
        in_copy(j).wait()
        if j >= 1:
            out_copy(j - 1).wait()
        if j + 1 < _NCHUNK:
            in_copy(j + 1).start()
        out_copy(j).start()
    out_copy(_NCHUNK - 1).wait()


def kernel(x):
    run = functools.partial(
        pl.kernel,
        mesh=plsc.VectorSubcoreMesh(core_axis_name="c", subcore_axis_name="s"),
        out_type=jax.ShapeDtypeStruct(x.shape, x.dtype),
        scratch_types=[
            pltpu.VMEM((_W0, _C1, 56, 56), jnp.float32),
            pltpu.VMEM((_W0, _C1, 56, 56), jnp.float32),
            pltpu.SemaphoreType.DMA,
            pltpu.SemaphoreType.DMA,
            pltpu.SemaphoreType.DMA,
            pltpu.SemaphoreType.DMA,
        ],
    )(_sc_copy)
    return run(x)
